# Initial kernel scaffold; baseline (speedup 1.0000x reference)
import jax
import jax.numpy as jnp
from jax.experimental import pallas as pl


def kernel(z_e, embed):
    raise NotImplementedError("write your pallas kernel here")


# speedup vs baseline: 3.1182x; 3.1182x over previous
import functools

import jax
import jax.numpy as jnp
from jax import lax
from jax.experimental import pallas as pl
from jax.experimental.pallas import tpu as pltpu
from jax.experimental.pallas import tpu_sc as plsc

_B, _T, _D = 16, 576, 256
_N, _K = 4, 1024
_Dn = _D // _N
_BT = _B * _T
_TB = 1152
_GRID = _BT // _TB
_IC = 128
_NI = _TB // _IC
_NC, _NS = 2, 16

_PREC = lax.Precision.DEFAULT


def _tc_body(z_ref, emb_ref, amin_ref, gidx_ref, diff_ref, ppl_ref, counts_ref):
    i = pl.program_id(0)

    @pl.when(i == 0)
    def _init():
        counts_ref[...] = jnp.zeros_like(counts_ref)
        diff_ref[...] = jnp.zeros_like(diff_ref)
        ppl_ref[...] = jnp.zeros_like(ppl_ref)

    z = z_ref[...]
    min_sum = jnp.zeros((1, 1), jnp.float32)
    step_counts = []
    for n in range(_N):
        zn = z[:, n * _Dn:(n + 1) * _Dn]
        en = emb_ref[n]
        mm = lax.dot_general(zn, en, (((1,), (0,)), ((), ())),
                             precision=_PREC,
                             preferred_element_type=jnp.float32)
        znorm = jnp.sum(zn * zn, axis=1, keepdims=True)
        enorm = jnp.sum(en * en, axis=0, keepdims=True)
        dist = znorm - 2.0 * mm + enorm
        amin = jnp.argmin(dist, axis=1).astype(jnp.int32)
        dmin = jnp.min(dist, axis=1)
        amin_ref[0, n, :] = amin
        gidx_ref[0, n, :] = amin + n * _K
        onehot = (amin[:, None]
                  == lax.broadcasted_iota(jnp.int32, (1, _K), 1)
                  ).astype(jnp.float32)
        step_counts.append(jnp.sum(onehot, axis=0, keepdims=True))
        min_sum = min_sum + jnp.sum(dmin).reshape(1, 1)
    counts_ref[...] = counts_ref[...] + jnp.concatenate(step_counts, axis=0)
    diff_ref[...] = diff_ref[...] + min_sum

    @pl.when(i == _GRID - 1)
    def _finish():
        diff_ref[...] = diff_ref[...] * (1.0 / (_B * _T * _D))
        probs = counts_ref[...] * (1.0 / _BT)
        ent = -jnp.sum(probs * jnp.log(probs + 1e-10), axis=-1)
        ppl_ref[...] = jnp.mean(jnp.exp(ent)).reshape(1, 1)


@functools.cache
def _sc_gather_kernel():
    mesh = plsc.VectorSubcoreMesh(core_axis_name="c", subcore_axis_name="s")

    @functools.partial(
        pl.kernel,
        out_type=jax.ShapeDtypeStruct((_N, _BT, _Dn), jnp.float32),
        mesh=mesh,
        scratch_types=[
            pltpu.VMEM((_TB,), jnp.int32),
            pltpu.VMEM((_TB, _Dn), jnp.float32),
            pltpu.SemaphoreType.DMA,
        ],
        compiler_params=pltpu.CompilerParams(use_tc_tiling_on_sc=False),
    )
    def _sc_gather(gidx_hbm, table_hbm, out_hbm, idx_v, rows_v, sem):
        c = lax.axis_index("c")
        s = lax.axis_index("s")
        w = s * _NC + c
        g = w // _N
        n = w % _N
        pltpu.sync_copy(gidx_hbm.at[g, n], idx_v)
        copies = []
        for j in range(_NI):
            copies.append(pltpu.async_copy(
                table_hbm.at[idx_v.at[pl.ds(j * _IC, _IC)]],
                rows_v.at[pl.ds(j * _IC, _IC)], sem))
        for cp in copies:
            cp.wait()
        pltpu.sync_copy(rows_v, out_hbm.at[n, pl.ds(g * _TB, _TB)])

    return _sc_gather


def _tc_stage(zflat, embed):
    return pl.pallas_call(
        _tc_body,
        grid=(_GRID,),
        in_specs=[
            pl.BlockSpec((_TB, _D), lambda i: (i, 0)),
            pl.BlockSpec((_N, _Dn, _K), lambda i: (0, 0, 0)),
        ],
        out_specs=[
            pl.BlockSpec((1, _N, _TB), lambda i: (i, 0, 0)),
            pl.BlockSpec((1, _N, _TB), lambda i: (i, 0, 0)),
            pl.BlockSpec((1, 1), lambda i: (0, 0)),
            pl.BlockSpec((1, 1), lambda i: (0, 0)),
        ],
        out_shape=[
            jax.ShapeDtypeStruct((_GRID, _N, _TB), jnp.int32),
            jax.ShapeDtypeStruct((_GRID, _N, _TB), jnp.int32),
            jax.ShapeDtypeStruct((1, 1), jnp.float32),
            jax.ShapeDtypeStruct((1, 1), jnp.float32),
        ],
        scratch_shapes=[pltpu.VMEM((_N, _K), jnp.float32)],
    )(zflat, embed)


def kernel(z_e, embed):
    zflat = z_e.reshape(_BT, _D)
    amin3, gidx3, diff, ppl = _tc_stage(zflat, embed)
    codes = jnp.transpose(embed, (0, 2, 1)).reshape(_N * _K, _Dn)
    quant = _sc_gather_kernel()(gidx3, codes)
    z_q = jnp.transpose(quant.reshape(_N, _B, _T, _Dn),
                        (1, 2, 0, 3)).reshape(_B, _T, _D)
    argmin = jnp.transpose(amin3, (1, 0, 2)).reshape(_N, _BT)
    return z_q, diff.reshape(()), ppl.reshape(()), argmin
